# baseline (speedup 1.0000x reference)
import functools

import jax
import jax.numpy as jnp
from jax import lax
from jax.experimental import pallas as pl
from jax.experimental.pallas import tpu as pltpu
from jax.experimental.pallas import tpu_sc as plsc

N = 10000
E = 320000
D = 128
NC = 2
NS = 16
NW = NC * NS
EPW = E // NW
CH = 80
NCH = EPW // CH
NP = 10240
RPT = NP // NS

_mesh = plsc.VectorSubcoreMesh(
    core_axis_name="c", subcore_axis_name="s", num_cores=NC, num_subcores=NS)


@functools.partial(
    pl.kernel,
    out_type=jax.ShapeDtypeStruct((NC, NP, D), jnp.float32),
    mesh=_mesh,
    scratch_types=[
        pltpu.VMEM((CH,), jnp.int32),
        pltpu.VMEM((CH,), jnp.int32),
        pltpu.VMEM((CH, D), jnp.float32),
        pltpu.VMEM_SHARED((NP, D), jnp.float32),
        pltpu.SemaphoreType.DMA,
    ],
)
def _scatter_rows1(src_hbm, dst_hbm, x_hbm, zrow_hbm, acc_out,
                   src_v, dst_v, rows_v, acc_s, sem):
    cid = lax.axis_index("c")
    sid = lax.axis_index("s")
    scbase = cid * NS * EPW

    pltpu.sync_copy(zrow_hbm, acc_s.at[pl.ds(sid * RPT, RPT)])
    plsc.subcore_barrier()

    @pl.when(sid == 0)
    def _():
        def body(c, carry):
            off = scbase + c * CH
            pltpu.sync_copy(src_hbm.at[pl.ds(off, CH)], src_v)
            pltpu.sync_copy(dst_hbm.at[pl.ds(off, CH)], dst_v)
            pltpu.async_copy(x_hbm.at[src_v], rows_v, sem).wait()
            pltpu.sync_copy(rows_v, acc_s.at[dst_v], add=True)
            return carry
        lax.fori_loop(0, NCH * NS, body, 0)

    plsc.subcore_barrier()
    pltpu.sync_copy(acc_s.at[pl.ds(sid * RPT, RPT)],
                    acc_out.at[cid, pl.ds(sid * RPT, RPT)])


def kernel(x, edge_index, W1, b1, W2, b2, W3, b3):
    src = edge_index[0].astype(jnp.int32)
    dst = edge_index[1].astype(jnp.int32)
    zrow = jnp.zeros((RPT, D), jnp.float32)
    acc = _scatter_rows1(src, dst, x, zrow)
    s = (acc[0] + acc[1])[:N]
    deg = jax.ops.segment_sum(jnp.ones((E,), jnp.float32), dst, num_segments=N)
    invd = 1.0 / jnp.maximum(deg, 1.0)[:, None]
    gate = (deg > 0.0).astype(jnp.float32)[:, None]
    h1 = (s * invd) @ W1[:D] + gate * (x @ W1[D:] + b1)
    h2 = x @ W2[:D] + h1 @ W2[D:] + b2
    a = h2 @ W3[:D]
    b = h2 @ W3[D:] + b3
    return a[src] + b[dst]


# speedup vs baseline: 7.0269x; 7.0269x over previous
import functools

import jax
import jax.numpy as jnp
from jax import lax
from jax.experimental import pallas as pl
from jax.experimental.pallas import tpu as pltpu
from jax.experimental.pallas import tpu_sc as plsc

N = 10000
E = 320000
D = 128
NC = 2
NS = 16
NW = NC * NS
EPW = E // NW
CH = 80
NCH = EPW // CH
NP = 10240
RPT = NP // NS

_mesh = plsc.VectorSubcoreMesh(
    core_axis_name="c", subcore_axis_name="s", num_cores=NC, num_subcores=NS)


@functools.partial(
    pl.kernel,
    out_type=(jax.ShapeDtypeStruct((NC, NP, D), jnp.float32),
              jax.ShapeDtypeStruct((NC, NP, D), jnp.float32)),
    mesh=_mesh,
    scratch_types=[
        pltpu.VMEM((CH,), jnp.int32),
        pltpu.VMEM((CH,), jnp.int32),
        pltpu.VMEM((CH, D), jnp.float32),
        pltpu.VMEM((CH, D), jnp.float32),
        pltpu.VMEM_SHARED((NP, D), jnp.float32),
        pltpu.SemaphoreType.DMA,
    ],
)
def _scatter_phase(src_hbm, dst_hbm, x_hbm, zrow_hbm, ones_hbm,
                   acc_out, deg_out, src_v, dst_v, rows_v, ones_v,
                   acc_s, sem):
    cid = lax.axis_index("c")
    sid = lax.axis_index("s")
    base = (cid * NS + sid) * EPW

    pltpu.sync_copy(zrow_hbm, rows_v)
    for j in range(RPT // CH):
        o = sid * RPT + j * CH
        pltpu.sync_copy(rows_v, acc_s.at[pl.ds(o, CH)])
    pltpu.sync_copy(ones_hbm, ones_v)
    plsc.subcore_barrier()

    def body(c, carry):
        off = base + c * CH
        pltpu.sync_copy(src_hbm.at[pl.ds(off, CH)], src_v)
        pltpu.sync_copy(dst_hbm.at[pl.ds(off, CH)], dst_v)
        cp = pltpu.async_copy(x_hbm.at[src_v], rows_v, sem)
        cp.wait()
        pltpu.sync_copy(rows_v, acc_s.at[dst_v], add=True)
        return carry

    lax.fori_loop(0, NCH, body, 0)
    plsc.subcore_barrier()

    for j in range(RPT // CH):
        o = sid * RPT + j * CH
        pltpu.sync_copy(acc_s.at[pl.ds(o, CH)], rows_v)
        pltpu.sync_copy(rows_v, acc_out.at[cid, pl.ds(o, CH)])
    pltpu.sync_copy(zrow_hbm, rows_v)
    for j in range(RPT // CH):
        o = sid * RPT + j * CH
        pltpu.sync_copy(rows_v, acc_s.at[pl.ds(o, CH)])
    plsc.subcore_barrier()

    def dbody(c, carry):
        off = base + c * CH
        pltpu.sync_copy(dst_hbm.at[pl.ds(off, CH)], dst_v)
        pltpu.sync_copy(ones_v, acc_s.at[dst_v], add=True)
        return carry

    lax.fori_loop(0, NCH, dbody, 0)
    plsc.subcore_barrier()

    for j in range(RPT // CH):
        o = sid * RPT + j * CH
        pltpu.sync_copy(acc_s.at[pl.ds(o, CH)], rows_v)
        pltpu.sync_copy(rows_v, deg_out.at[cid, pl.ds(o, CH)])


_BN = 1024


def _node_body(acc_ref, deg_ref, x_ref, w1_ref, b1_ref, w2_ref, b2_ref,
               w3_ref, b3_ref, a_ref, b_ref):
    s = acc_ref[0] + acc_ref[1]
    deg = (deg_ref[0] + deg_ref[1])[:, :1]
    invd = 1.0 / jnp.maximum(deg, 1.0)
    gate = (deg > 0.0).astype(jnp.float32)
    x = x_ref[...]
    h1 = (jnp.dot(s * invd, w1_ref[:D], preferred_element_type=jnp.float32)
          + gate * (jnp.dot(x, w1_ref[D:], preferred_element_type=jnp.float32)
                    + b1_ref[...]))
    h2 = (jnp.dot(x, w2_ref[:D], preferred_element_type=jnp.float32)
          + jnp.dot(h1, w2_ref[D:], preferred_element_type=jnp.float32)
          + b2_ref[...])
    a_ref[...] = jnp.dot(h2, w3_ref[:D], preferred_element_type=jnp.float32)
    b_ref[...] = (jnp.dot(h2, w3_ref[D:], preferred_element_type=jnp.float32)
                  + b3_ref[...])


_node_phase = pl.pallas_call(
    _node_body,
    grid=(NP // _BN,),
    in_specs=[
        pl.BlockSpec((NC, _BN, D), lambda i: (0, i, 0)),
        pl.BlockSpec((NC, _BN, D), lambda i: (0, i, 0)),
        pl.BlockSpec((_BN, D), lambda i: (i, 0)),
        pl.BlockSpec((2 * D, D), lambda i: (0, 0)),
        pl.BlockSpec((1, D), lambda i: (0, 0)),
        pl.BlockSpec((2 * D, D), lambda i: (0, 0)),
        pl.BlockSpec((1, D), lambda i: (0, 0)),
        pl.BlockSpec((2 * D, D), lambda i: (0, 0)),
        pl.BlockSpec((1, D), lambda i: (0, 0)),
    ],
    out_specs=[
        pl.BlockSpec((_BN, D), lambda i: (i, 0)),
        pl.BlockSpec((_BN, D), lambda i: (i, 0)),
    ],
    out_shape=[
        jax.ShapeDtypeStruct((NP, D), jnp.float32),
        jax.ShapeDtypeStruct((NP, D), jnp.float32),
    ],
)


@functools.partial(
    pl.kernel,
    out_type=jax.ShapeDtypeStruct((E, D), jnp.float32),
    mesh=_mesh,
    scratch_types=[
        pltpu.VMEM((CH,), jnp.int32),
        pltpu.VMEM((CH,), jnp.int32),
        pltpu.VMEM((CH, D), jnp.float32),
        pltpu.VMEM((CH, D), jnp.float32),
        pltpu.SemaphoreType.DMA,
        pltpu.SemaphoreType.DMA,
    ],
)
def _gather_phase(src_hbm, dst_hbm, a_hbm, b_hbm, ef_hbm,
                  src_v, dst_v, a_v, b_v, sem_a, sem_b):
    cid = lax.axis_index("c")
    sid = lax.axis_index("s")
    base = (cid * NS + sid) * EPW

    def body(c, carry):
        off = base + c * CH
        pltpu.sync_copy(src_hbm.at[pl.ds(off, CH)], src_v)
        pltpu.sync_copy(dst_hbm.at[pl.ds(off, CH)], dst_v)
        cp_a = pltpu.async_copy(a_hbm.at[src_v], a_v, sem_a)
        cp_b = pltpu.async_copy(b_hbm.at[dst_v], b_v, sem_b)
        cp_a.wait()
        cp_b.wait()

        def add_row(i, carry2):
            for j in range(D // 16):
                sl = pl.ds(j * 16, 16)
                a_v[i, sl] = a_v[i, sl] + b_v[i, sl]
            return carry2

        lax.fori_loop(0, CH, add_row, 0)
        pltpu.sync_copy(a_v, ef_hbm.at[pl.ds(off, CH)])
        return carry

    lax.fori_loop(0, NCH, body, 0)


def kernel(x, edge_index, W1, b1, W2, b2, W3, b3):
    src = edge_index[0].astype(jnp.int32)
    dst = edge_index[1].astype(jnp.int32)
    zrow = jnp.zeros((CH, D), jnp.float32)
    ones = jnp.ones((CH, D), jnp.float32)
    acc, deg = _scatter_phase(src, dst, x, zrow, ones)
    xp = jnp.pad(x, ((0, NP - N), (0, 0)))
    a, b = _node_phase(acc, deg, xp, W1, b1.reshape(1, D), W2,
                       b2.reshape(1, D), W3, b3.reshape(1, D))
    return _gather_phase(src, dst, a, b)
